# baseline (speedup 1.0000x reference)
import functools

import jax
import jax.numpy as jnp
from jax import lax
from jax.experimental import pallas as pl
from jax.experimental.pallas import tpu as pltpu
from jax.experimental.pallas import tpu_sc as plsc

N_ROWS = 320000
D_FEAT = 128
NUM_SEGMENTS = 10000

NC = 2
NS = 16
NW = NC * NS
ROWS_PER_W = N_ROWS // NW
BLK = 80
NBLK = ROWS_PER_W // BLK
OCHUNK = 80
NOCHUNK = NUM_SEGMENTS // OCHUNK
OITER = -(-NOCHUNK // NS)


def _sc_partial_sums(X, keys):
    mesh = plsc.VectorSubcoreMesh(core_axis_name="c", subcore_axis_name="s")

    @functools.partial(
        pl.kernel,
        out_type=jax.ShapeDtypeStruct((NC, NUM_SEGMENTS, D_FEAT), jnp.float32),
        mesh=mesh,
        scratch_types=[
            pltpu.VMEM((BLK, D_FEAT), jnp.float32),
            pltpu.VMEM((BLK, D_FEAT), jnp.float32),
            pltpu.VMEM((BLK,), jnp.int32),
            pltpu.VMEM((BLK,), jnp.int32),
            pltpu.VMEM((OCHUNK, D_FEAT), jnp.float32),
            pltpu.VMEM_SHARED((NUM_SEGMENTS, D_FEAT), jnp.float32),
            pltpu.SemaphoreType.DMA,
            pltpu.SemaphoreType.DMA,
            pltpu.SemaphoreType.DMA,
            pltpu.SemaphoreType.DMA,
        ],
    )
    def k(x_hbm, keys_hbm, out_hbm, xbuf_a, xbuf_b, kbuf_a, kbuf_b,
          zbuf, acc, sem_a, sem_b, sem_sa, sem_sb):
        c = lax.axis_index("c")
        s = lax.axis_index("s")
        wid = c * NS + s

        @pl.loop(0, OCHUNK)
        def _(r):
            @pl.loop(0, D_FEAT, step=16)
            def _(col):
                zbuf[r, pl.ds(col, 16)] = jnp.zeros((16,), jnp.float32)

        @pl.loop(0, OITER)
        def _(j):
            chunk = s + NS * j

            @pl.when(chunk < NOCHUNK)
            def _():
                pltpu.sync_copy(zbuf, acc.at[pl.ds(chunk * OCHUNK, OCHUNK)])

        plsc.subcore_barrier()

        base = wid * ROWS_PER_W

        def kslc(i):
            return keys_hbm.at[pl.ds(base + i * BLK, BLK)]

        def xslc(i):
            return x_hbm.at[pl.ds(base + i * BLK, BLK)]

        def start_load(i, xbuf, kbuf, sem):
            pltpu.async_copy(xslc(i), xbuf, sem)
            pltpu.async_copy(kslc(i), kbuf, sem)

        def wait_load(i, xbuf, kbuf, sem):
            pltpu.make_async_copy(xslc(i), xbuf, sem).wait()
            pltpu.make_async_copy(kslc(i), kbuf, sem).wait()

        start_load(0, xbuf_a, kbuf_a, sem_a)
        start_load(1, xbuf_b, kbuf_b, sem_b)

        @pl.loop(0, NBLK // 2)
        def _(j):
            i0 = 2 * j
            wait_load(i0, xbuf_a, kbuf_a, sem_a)
            scat_a = pltpu.async_copy(
                xbuf_a, acc.at[kbuf_a], sem_sa, add=True
            )
            wait_load(i0 + 1, xbuf_b, kbuf_b, sem_b)
            scat_b = pltpu.async_copy(
                xbuf_b, acc.at[kbuf_b], sem_sb, add=True
            )
            scat_a.wait()
            start_load(i0 + 2, xbuf_a, kbuf_a, sem_a)
            scat_b.wait()

            @pl.when(i0 + 3 < NBLK)
            def _():
                start_load(i0 + 3, xbuf_b, kbuf_b, sem_b)

        wait_load(NBLK - 1, xbuf_a, kbuf_a, sem_a)
        pltpu.sync_copy(xbuf_a, acc.at[kbuf_a], add=True)

        plsc.subcore_barrier()

        @pl.loop(0, OITER)
        def _(j):
            chunk = s + NS * j

            @pl.when(chunk < NOCHUNK)
            def _():
                pltpu.sync_copy(
                    acc.at[pl.ds(chunk * OCHUNK, OCHUNK)],
                    out_hbm.at[c, pl.ds(chunk * OCHUNK, OCHUNK)],
                )

    return k(X, keys)


def _tc_combine(a, b):
    def body(a_ref, b_ref, o_ref):
        o_ref[...] = a_ref[...] + b_ref[...]

    return pl.pallas_call(
        body,
        grid=(10,),
        in_specs=[
            pl.BlockSpec((1000, D_FEAT), lambda i: (i, 0)),
            pl.BlockSpec((1000, D_FEAT), lambda i: (i, 0)),
        ],
        out_specs=pl.BlockSpec((1000, D_FEAT), lambda i: (i, 0)),
        out_shape=jax.ShapeDtypeStruct((NUM_SEGMENTS, D_FEAT), jnp.float32),
    )(a, b)


@jax.jit
def kernel(X, keys):
    keys = keys.astype(jnp.int32)
    acc = _sc_partial_sums(X, keys)
    return _tc_combine(acc[0], acc[1])


# speedup vs baseline: 1.2891x; 1.2891x over previous
import functools

import jax
import jax.numpy as jnp
from jax import lax
from jax.experimental import pallas as pl
from jax.experimental.pallas import tpu as pltpu
from jax.experimental.pallas import tpu_sc as plsc

N_ROWS = 320000
D_FEAT = 128
NUM_SEGMENTS = 10000

NC = 2
NS = 16
NW = NC * NS
ROWS_PER_W = N_ROWS // NW
BLK = 128
NFULL = ROWS_PER_W // BLK
TAIL = ROWS_PER_W - NFULL * BLK
NBUF = 2
ZROWS = 16
OCHUNK = 80
NOCHUNK = NUM_SEGMENTS // OCHUNK
OITER = -(-NOCHUNK // NS)


def _sc_partial_sums(X, keys):
    mesh = plsc.VectorSubcoreMesh(core_axis_name="c", subcore_axis_name="s")

    @functools.partial(
        pl.kernel,
        out_type=jax.ShapeDtypeStruct((NC, NUM_SEGMENTS, D_FEAT), jnp.float32),
        mesh=mesh,
        scratch_types=[
            pltpu.VMEM((BLK, D_FEAT), jnp.float32),
            pltpu.VMEM((BLK, D_FEAT), jnp.float32),
            pltpu.VMEM((BLK,), jnp.int32),
            pltpu.VMEM((BLK,), jnp.int32),
            pltpu.VMEM((TAIL, D_FEAT), jnp.float32),
            pltpu.VMEM((TAIL,), jnp.int32),
            pltpu.VMEM((ZROWS, D_FEAT), jnp.float32),
            pltpu.VMEM_SHARED((NUM_SEGMENTS, D_FEAT), jnp.float32),
            pltpu.SemaphoreType.DMA,
            pltpu.SemaphoreType.DMA,
            pltpu.SemaphoreType.DMA,
        ],
    )
    def k(x_hbm, keys_hbm, out_hbm, xbuf_a, xbuf_b,
          kbuf_a, kbuf_b, xbuf_t, kbuf_t,
          zbuf, acc, sem_a, sem_b, sem_t):
        c = lax.axis_index("c")
        s = lax.axis_index("s")
        wid = c * NS + s

        @pl.loop(0, ZROWS)
        def _(r):
            @pl.loop(0, D_FEAT, step=16)
            def _(col):
                zbuf[r, pl.ds(col, 16)] = jnp.zeros((16,), jnp.float32)

        @pl.loop(0, OITER)
        def _(j):
            chunk = s + NS * j

            @pl.when(chunk < NOCHUNK)
            def _():
                @pl.loop(0, OCHUNK // ZROWS)
                def _(j2):
                    pltpu.sync_copy(
                        zbuf,
                        acc.at[pl.ds(chunk * OCHUNK + j2 * ZROWS, ZROWS)],
                    )

        plsc.subcore_barrier()

        base = wid * ROWS_PER_W

        def kslc(i):
            return keys_hbm.at[pl.ds(base + i * BLK, BLK)]

        def xslc(i):
            return x_hbm.at[pl.ds(base + i * BLK, BLK)]

        def start_load(i, xbuf, kbuf, sem):
            pltpu.async_copy(xslc(i), xbuf, sem)
            pltpu.async_copy(kslc(i), kbuf, sem)

        def wait_load(i, xbuf, kbuf, sem):
            pltpu.make_async_copy(xslc(i), xbuf, sem).wait()
            pltpu.make_async_copy(kslc(i), kbuf, sem).wait()

        bufs = ((xbuf_a, kbuf_a, sem_a), (xbuf_b, kbuf_b, sem_b))

        def refill(i, xbuf, kbuf, sem):
            @pl.when(i + NBUF < NFULL)
            def _():
                start_load(i + NBUF, xbuf, kbuf, sem)

        for b in range(NBUF):
            start_load(b, *bufs[b])
        toff = base + NFULL * BLK
        pltpu.async_copy(x_hbm.at[pl.ds(toff, TAIL)], xbuf_t, sem_t)
        pltpu.async_copy(keys_hbm.at[pl.ds(toff, TAIL)], kbuf_t, sem_t)

        @pl.loop(0, NFULL // NBUF)
        def _(g):
            for b in range(NBUF):
                i = NBUF * g + b
                wait_load(i, *bufs[b])
                pltpu.sync_copy(bufs[b][0], acc.at[bufs[b][1]], add=True)
                refill(i, *bufs[b])

        pltpu.make_async_copy(x_hbm.at[pl.ds(toff, TAIL)], xbuf_t,
                              sem_t).wait()
        pltpu.make_async_copy(keys_hbm.at[pl.ds(toff, TAIL)], kbuf_t,
                              sem_t).wait()
        pltpu.sync_copy(xbuf_t, acc.at[kbuf_t], add=True)

        plsc.subcore_barrier()

        @pl.loop(0, OITER)
        def _(j):
            chunk = s + NS * j

            @pl.when(chunk < NOCHUNK)
            def _():
                pltpu.sync_copy(
                    acc.at[pl.ds(chunk * OCHUNK, OCHUNK)],
                    out_hbm.at[c, pl.ds(chunk * OCHUNK, OCHUNK)],
                )

    return k(X, keys)


def _tc_combine(a, b):
    def body(a_ref, b_ref, o_ref):
        o_ref[...] = a_ref[...] + b_ref[...]

    return pl.pallas_call(
        body,
        grid=(10,),
        in_specs=[
            pl.BlockSpec((1000, D_FEAT), lambda i: (i, 0)),
            pl.BlockSpec((1000, D_FEAT), lambda i: (i, 0)),
        ],
        out_specs=pl.BlockSpec((1000, D_FEAT), lambda i: (i, 0)),
        out_shape=jax.ShapeDtypeStruct((NUM_SEGMENTS, D_FEAT), jnp.float32),
    )(a, b)


@jax.jit
def kernel(X, keys):
    keys = keys.astype(jnp.int32)
    acc = _sc_partial_sums(X, keys)
    return _tc_combine(acc[0], acc[1])


# speedup vs baseline: 1.3193x; 1.0234x over previous
import functools

import jax
import jax.numpy as jnp
from jax import lax
from jax.experimental import pallas as pl
from jax.experimental.pallas import tpu as pltpu
from jax.experimental.pallas import tpu_sc as plsc

N_ROWS = 320000
D_FEAT = 128
NUM_SEGMENTS = 10000

NC = 2
NS = 16
NW = NC * NS
ROWS_PER_W = N_ROWS // NW
BLK = 128
NFULL = ROWS_PER_W // BLK
TAIL = ROWS_PER_W - NFULL * BLK
NBUF = 2
ZROWS = 16
WSTRIPE = 640
WLAST = NUM_SEGMENTS - (NS - 1) * WSTRIPE
OCHUNK = 80
NOCHUNK = NUM_SEGMENTS // OCHUNK
OITER = -(-NOCHUNK // NS)


def _sc_partial_sums(X, keys):
    mesh = plsc.VectorSubcoreMesh(core_axis_name="c", subcore_axis_name="s")

    @functools.partial(
        pl.kernel,
        out_type=jax.ShapeDtypeStruct((NC, NUM_SEGMENTS, D_FEAT), jnp.float32),
        mesh=mesh,
        scratch_types=[
            pltpu.VMEM((BLK, D_FEAT), jnp.float32),
            pltpu.VMEM((BLK, D_FEAT), jnp.float32),
            pltpu.VMEM((BLK,), jnp.int32),
            pltpu.VMEM((BLK,), jnp.int32),
            pltpu.VMEM((TAIL, D_FEAT), jnp.float32),
            pltpu.VMEM((TAIL,), jnp.int32),
            pltpu.VMEM((ZROWS, D_FEAT), jnp.float32),
            pltpu.VMEM_SHARED((NUM_SEGMENTS, D_FEAT), jnp.float32),
            pltpu.SemaphoreType.DMA,
            pltpu.SemaphoreType.DMA,
            pltpu.SemaphoreType.DMA,
            pltpu.SemaphoreType.DMA,
        ],
    )
    def k(x_hbm, keys_hbm, out_hbm, xbuf_a, xbuf_b,
          kbuf_a, kbuf_b, xbuf_t, kbuf_t,
          zbuf, acc, sem_a, sem_b, sem_t, sem_z):
        c = lax.axis_index("c")
        s = lax.axis_index("s")
        wid = c * NS + s

        base = wid * ROWS_PER_W

        def kslc(i):
            return keys_hbm.at[pl.ds(base + i * BLK, BLK)]

        def xslc(i):
            return x_hbm.at[pl.ds(base + i * BLK, BLK)]

        def start_load(i, xbuf, kbuf, sem):
            pltpu.async_copy(xslc(i), xbuf, sem)
            pltpu.async_copy(kslc(i), kbuf, sem)

        def wait_load(i, xbuf, kbuf, sem):
            pltpu.make_async_copy(xslc(i), xbuf, sem).wait()
            pltpu.make_async_copy(kslc(i), kbuf, sem).wait()

        bufs = ((xbuf_a, kbuf_a, sem_a), (xbuf_b, kbuf_b, sem_b))

        def refill(i, xbuf, kbuf, sem):
            @pl.when(i + NBUF < NFULL)
            def _():
                start_load(i + NBUF, xbuf, kbuf, sem)

        for b in range(NBUF):
            start_load(b, *bufs[b])
        toff = base + NFULL * BLK
        pltpu.async_copy(x_hbm.at[pl.ds(toff, TAIL)], xbuf_t, sem_t)
        pltpu.async_copy(keys_hbm.at[pl.ds(toff, TAIL)], kbuf_t, sem_t)

        @pl.loop(0, ZROWS)
        def _(r):
            @pl.loop(0, D_FEAT, step=16)
            def _(col):
                zbuf[r, pl.ds(col, 16)] = jnp.zeros((16,), jnp.float32)

        def each_zero_chunk(fn):
            @pl.loop(0, OITER)
            def _(j):
                chunk = s + NS * j

                @pl.when(chunk < NOCHUNK)
                def _():
                    @pl.loop(0, OCHUNK // ZROWS)
                    def _(j2):
                        fn(pl.ds(chunk * OCHUNK + j2 * ZROWS, ZROWS))

        each_zero_chunk(
            lambda d: pltpu.async_copy(zbuf, acc.at[d], sem_z))
        each_zero_chunk(
            lambda d: pltpu.make_async_copy(zbuf, acc.at[d], sem_z).wait())

        plsc.subcore_barrier()

        @pl.loop(0, NFULL // NBUF)
        def _(g):
            for b in range(NBUF):
                i = NBUF * g + b
                wait_load(i, *bufs[b])
                pltpu.sync_copy(bufs[b][0], acc.at[bufs[b][1]], add=True)
                refill(i, *bufs[b])

        pltpu.make_async_copy(x_hbm.at[pl.ds(toff, TAIL)], xbuf_t,
                              sem_t).wait()
        pltpu.make_async_copy(keys_hbm.at[pl.ds(toff, TAIL)], kbuf_t,
                              sem_t).wait()
        pltpu.sync_copy(xbuf_t, acc.at[kbuf_t], add=True)

        plsc.subcore_barrier()

        @pl.when(s < NS - 1)
        def _():
            pltpu.sync_copy(
                acc.at[pl.ds(s * WSTRIPE, WSTRIPE)],
                out_hbm.at[c, pl.ds(s * WSTRIPE, WSTRIPE)],
            )

        @pl.when(s == NS - 1)
        def _():
            pltpu.sync_copy(
                acc.at[pl.ds((NS - 1) * WSTRIPE, WLAST)],
                out_hbm.at[c, pl.ds((NS - 1) * WSTRIPE, WLAST)],
            )

    return k(X, keys)


def _tc_combine(a, b):
    def body(a_ref, b_ref, o_ref):
        o_ref[...] = a_ref[...] + b_ref[...]

    return pl.pallas_call(
        body,
        grid=(10,),
        in_specs=[
            pl.BlockSpec((1000, D_FEAT), lambda i: (i, 0)),
            pl.BlockSpec((1000, D_FEAT), lambda i: (i, 0)),
        ],
        out_specs=pl.BlockSpec((1000, D_FEAT), lambda i: (i, 0)),
        out_shape=jax.ShapeDtypeStruct((NUM_SEGMENTS, D_FEAT), jnp.float32),
    )(a, b)


@jax.jit
def kernel(X, keys):
    keys = keys.astype(jnp.int32)
    acc = _sc_partial_sums(X, keys)
    return _tc_combine(acc[0], acc[1])


# speedup vs baseline: 1.3961x; 1.0582x over previous
import functools

import jax
import jax.numpy as jnp
from jax import lax
from jax.experimental import pallas as pl
from jax.experimental.pallas import tpu as pltpu
from jax.experimental.pallas import tpu_sc as plsc

N_ROWS = 320000
D_FEAT = 128
NUM_SEGMENTS = 10000

NC = 2
NS = 16
NW = NC * NS
ROWS_PER_W = N_ROWS // NW
BLK = 64
NFULL = ROWS_PER_W // BLK
TAIL = ROWS_PER_W - NFULL * BLK
NBUF = 4
ZROWS = 16
WSTRIPE = 640
WLAST = NUM_SEGMENTS - (NS - 1) * WSTRIPE
OCHUNK = 80
NOCHUNK = NUM_SEGMENTS // OCHUNK
OITER = -(-NOCHUNK // NS)


def _sc_partial_sums(X, keys):
    mesh = plsc.VectorSubcoreMesh(core_axis_name="c", subcore_axis_name="s")

    @functools.partial(
        pl.kernel,
        out_type=jax.ShapeDtypeStruct((NC, NUM_SEGMENTS, D_FEAT), jnp.float32),
        mesh=mesh,
        scratch_types=[
            pltpu.VMEM((BLK, D_FEAT), jnp.float32),
            pltpu.VMEM((BLK, D_FEAT), jnp.float32),
            pltpu.VMEM((BLK, D_FEAT), jnp.float32),
            pltpu.VMEM((BLK, D_FEAT), jnp.float32),
            pltpu.VMEM((BLK,), jnp.int32),
            pltpu.VMEM((BLK,), jnp.int32),
            pltpu.VMEM((BLK,), jnp.int32),
            pltpu.VMEM((BLK,), jnp.int32),
            pltpu.VMEM((TAIL, D_FEAT), jnp.float32),
            pltpu.VMEM((TAIL,), jnp.int32),
            pltpu.VMEM((ZROWS, D_FEAT), jnp.float32),
            pltpu.VMEM_SHARED((NUM_SEGMENTS, D_FEAT), jnp.float32),
            pltpu.SemaphoreType.DMA,
            pltpu.SemaphoreType.DMA,
            pltpu.SemaphoreType.DMA,
            pltpu.SemaphoreType.DMA,
            pltpu.SemaphoreType.DMA,
            pltpu.SemaphoreType.DMA,
        ],
    )
    def k(x_hbm, keys_hbm, out_hbm, xbuf_a, xbuf_b, xbuf_c, xbuf_d,
          kbuf_a, kbuf_b, kbuf_c, kbuf_d, xbuf_t, kbuf_t,
          zbuf, acc, sem_a, sem_b, sem_c, sem_d, sem_t, sem_z):
        c = lax.axis_index("c")
        s = lax.axis_index("s")
        wid = c * NS + s

        base = wid * ROWS_PER_W

        def kslc(i):
            return keys_hbm.at[pl.ds(base + i * BLK, BLK)]

        def xslc(i):
            return x_hbm.at[pl.ds(base + i * BLK, BLK)]

        def start_load(i, xbuf, kbuf, sem):
            pltpu.async_copy(xslc(i), xbuf, sem)
            pltpu.async_copy(kslc(i), kbuf, sem)

        def wait_load(i, xbuf, kbuf, sem):
            pltpu.make_async_copy(xslc(i), xbuf, sem).wait()
            pltpu.make_async_copy(kslc(i), kbuf, sem).wait()

        bufs = ((xbuf_a, kbuf_a, sem_a), (xbuf_b, kbuf_b, sem_b),
                (xbuf_c, kbuf_c, sem_c), (xbuf_d, kbuf_d, sem_d))

        def refill(i, xbuf, kbuf, sem):
            @pl.when(i + NBUF < NFULL)
            def _():
                start_load(i + NBUF, xbuf, kbuf, sem)

        for b in range(NBUF):
            start_load(b, *bufs[b])
        toff = base + NFULL * BLK
        pltpu.async_copy(x_hbm.at[pl.ds(toff, TAIL)], xbuf_t, sem_t)
        pltpu.async_copy(keys_hbm.at[pl.ds(toff, TAIL)], kbuf_t, sem_t)

        @pl.loop(0, ZROWS)
        def _(r):
            @pl.loop(0, D_FEAT, step=16)
            def _(col):
                zbuf[r, pl.ds(col, 16)] = jnp.zeros((16,), jnp.float32)

        def each_zero_chunk(fn):
            @pl.loop(0, OITER)
            def _(j):
                chunk = s + NS * j

                @pl.when(chunk < NOCHUNK)
                def _():
                    @pl.loop(0, OCHUNK // ZROWS)
                    def _(j2):
                        fn(pl.ds(chunk * OCHUNK + j2 * ZROWS, ZROWS))

        each_zero_chunk(
            lambda d: pltpu.async_copy(zbuf, acc.at[d], sem_z))
        each_zero_chunk(
            lambda d: pltpu.make_async_copy(zbuf, acc.at[d], sem_z).wait())

        plsc.subcore_barrier()

        @pl.loop(0, NFULL // NBUF)
        def _(g):
            for b in range(NBUF):
                i = NBUF * g + b
                wait_load(i, *bufs[b])
                pltpu.sync_copy(bufs[b][0], acc.at[bufs[b][1]], add=True)
                refill(i, *bufs[b])

        pltpu.make_async_copy(x_hbm.at[pl.ds(toff, TAIL)], xbuf_t,
                              sem_t).wait()
        pltpu.make_async_copy(keys_hbm.at[pl.ds(toff, TAIL)], kbuf_t,
                              sem_t).wait()
        pltpu.sync_copy(xbuf_t, acc.at[kbuf_t], add=True)

        plsc.subcore_barrier()

        @pl.when(s < NS - 1)
        def _():
            pltpu.sync_copy(
                acc.at[pl.ds(s * WSTRIPE, WSTRIPE)],
                out_hbm.at[c, pl.ds(s * WSTRIPE, WSTRIPE)],
            )

        @pl.when(s == NS - 1)
        def _():
            pltpu.sync_copy(
                acc.at[pl.ds((NS - 1) * WSTRIPE, WLAST)],
                out_hbm.at[c, pl.ds((NS - 1) * WSTRIPE, WLAST)],
            )

    return k(X, keys)


def _tc_combine(a, b):
    def body(a_ref, b_ref, o_ref):
        o_ref[...] = a_ref[...] + b_ref[...]

    return pl.pallas_call(
        body,
        grid=(10,),
        in_specs=[
            pl.BlockSpec((1000, D_FEAT), lambda i: (i, 0)),
            pl.BlockSpec((1000, D_FEAT), lambda i: (i, 0)),
        ],
        out_specs=pl.BlockSpec((1000, D_FEAT), lambda i: (i, 0)),
        out_shape=jax.ShapeDtypeStruct((NUM_SEGMENTS, D_FEAT), jnp.float32),
    )(a, b)


@jax.jit
def kernel(X, keys):
    keys = keys.astype(jnp.int32)
    acc = _sc_partial_sums(X, keys)
    return _tc_combine(acc[0], acc[1])


# speedup vs baseline: 1.4716x; 1.0541x over previous
import functools

import jax
import jax.numpy as jnp
from jax import lax
from jax.experimental import pallas as pl
from jax.experimental.pallas import tpu as pltpu
from jax.experimental.pallas import tpu_sc as plsc

N_ROWS = 320000
D_FEAT = 128
NUM_SEGMENTS = 10000

NC = 2
NS = 16
NW = NC * NS
ROWS_PER_W = N_ROWS // NW
BLK = 64
NFULL = ROWS_PER_W // BLK
TAIL = ROWS_PER_W - NFULL * BLK
NBUF = 4
ZROWS = 16
WSTRIPE = 640
WLAST = NUM_SEGMENTS - (NS - 1) * WSTRIPE
OCHUNK = 80
NOCHUNK = NUM_SEGMENTS // OCHUNK
OITER = -(-NOCHUNK // NS)


def _sc_partial_sums(X, keys):
    mesh = plsc.VectorSubcoreMesh(core_axis_name="c", subcore_axis_name="s")

    @functools.partial(
        pl.kernel,
        out_type=jax.ShapeDtypeStruct((NC, NUM_SEGMENTS, D_FEAT), jnp.float32),
        mesh=mesh,
        scratch_types=[
            pltpu.VMEM((BLK, D_FEAT), jnp.float32),
            pltpu.VMEM((BLK, D_FEAT), jnp.float32),
            pltpu.VMEM((BLK, D_FEAT), jnp.float32),
            pltpu.VMEM((BLK, D_FEAT), jnp.float32),
            pltpu.VMEM((BLK,), jnp.int32),
            pltpu.VMEM((BLK,), jnp.int32),
            pltpu.VMEM((BLK,), jnp.int32),
            pltpu.VMEM((BLK,), jnp.int32),
            pltpu.VMEM((TAIL, D_FEAT), jnp.float32),
            pltpu.VMEM((TAIL,), jnp.int32),
            pltpu.VMEM((ZROWS, D_FEAT), jnp.float32),
            pltpu.VMEM_SHARED((NUM_SEGMENTS, D_FEAT), jnp.float32),
            pltpu.SemaphoreType.DMA,
            pltpu.SemaphoreType.DMA,
            pltpu.SemaphoreType.DMA,
            pltpu.SemaphoreType.DMA,
            pltpu.SemaphoreType.DMA,
            pltpu.SemaphoreType.DMA,
        ],
    )
    def k(x_hbm, keys_hbm, out_hbm, xbuf_a, xbuf_b, xbuf_c, xbuf_d,
          kbuf_a, kbuf_b, kbuf_c, kbuf_d, xbuf_t, kbuf_t,
          zbuf, acc, sem_a, sem_b, sem_c, sem_d, sem_t, sem_z):
        c = lax.axis_index("c")
        s = lax.axis_index("s")
        wid = c * NS + s

        base = wid * ROWS_PER_W

        def kslc(i):
            return keys_hbm.at[pl.ds(base + i * BLK, BLK)]

        def xslc(i):
            return x_hbm.at[pl.ds(base + i * BLK, BLK)]

        def start_load(i, xbuf, kbuf, sem):
            pltpu.async_copy(xslc(i), xbuf, sem)
            pltpu.async_copy(kslc(i), kbuf, sem)

        def wait_load(i, xbuf, kbuf, sem):
            pltpu.make_async_copy(xslc(i), xbuf, sem).wait()
            pltpu.make_async_copy(kslc(i), kbuf, sem).wait()

        bufs = ((xbuf_a, kbuf_a, sem_a), (xbuf_b, kbuf_b, sem_b),
                (xbuf_c, kbuf_c, sem_c), (xbuf_d, kbuf_d, sem_d))

        def refill(i, xbuf, kbuf, sem):
            @pl.when(i + NBUF < NFULL)
            def _():
                start_load(i + NBUF, xbuf, kbuf, sem)

        for b in range(NBUF):
            start_load(b, *bufs[b])
        toff = base + NFULL * BLK
        pltpu.async_copy(x_hbm.at[pl.ds(toff, TAIL)], xbuf_t, sem_t)
        pltpu.async_copy(keys_hbm.at[pl.ds(toff, TAIL)], kbuf_t, sem_t)

        @pl.loop(0, ZROWS)
        def _(r):
            @pl.loop(0, D_FEAT, step=16)
            def _(col):
                zbuf[r, pl.ds(col, 16)] = jnp.zeros((16,), jnp.float32)

        def each_zero_chunk(fn):
            @pl.loop(0, OITER)
            def _(j):
                chunk = s + NS * j

                @pl.when(chunk < NOCHUNK)
                def _():
                    @pl.loop(0, OCHUNK // ZROWS)
                    def _(j2):
                        fn(pl.ds(chunk * OCHUNK + j2 * ZROWS, ZROWS))

        each_zero_chunk(
            lambda d: pltpu.async_copy(zbuf, acc.at[d], sem_z))
        each_zero_chunk(
            lambda d: pltpu.make_async_copy(zbuf, acc.at[d], sem_z).wait())

        plsc.subcore_barrier()

        @pl.loop(0, NFULL // NBUF)
        def _(g):
            for b in range(NBUF):
                i = NBUF * g + b
                wait_load(i, *bufs[b])
                pltpu.sync_copy(bufs[b][0], acc.at[bufs[b][1]], add=True)
                refill(i, *bufs[b])

        pltpu.make_async_copy(x_hbm.at[pl.ds(toff, TAIL)], xbuf_t,
                              sem_t).wait()
        pltpu.make_async_copy(keys_hbm.at[pl.ds(toff, TAIL)], kbuf_t,
                              sem_t).wait()
        pltpu.sync_copy(xbuf_t, acc.at[kbuf_t], add=True)

        plsc.subcore_barrier()

        @pl.when(s < NS - 1)
        def _():
            pltpu.sync_copy(
                acc.at[pl.ds(s * WSTRIPE, WSTRIPE)],
                out_hbm.at[c, pl.ds(s * WSTRIPE, WSTRIPE)],
            )

        @pl.when(s == NS - 1)
        def _():
            pltpu.sync_copy(
                acc.at[pl.ds((NS - 1) * WSTRIPE, WLAST)],
                out_hbm.at[c, pl.ds((NS - 1) * WSTRIPE, WLAST)],
            )

    return k(X, keys)


def _tc_combine(acc):
    def body(a_ref, b_ref, o_ref):
        o_ref[...] = a_ref[0] + b_ref[0]

    return pl.pallas_call(
        body,
        grid=(10,),
        in_specs=[
            pl.BlockSpec((1, 1000, D_FEAT), lambda i: (0, i, 0)),
            pl.BlockSpec((1, 1000, D_FEAT), lambda i: (1, i, 0)),
        ],
        out_specs=pl.BlockSpec((1000, D_FEAT), lambda i: (i, 0)),
        out_shape=jax.ShapeDtypeStruct((NUM_SEGMENTS, D_FEAT), jnp.float32),
    )(acc, acc)


@jax.jit
def kernel(X, keys):
    keys = keys.astype(jnp.int32)
    acc = _sc_partial_sums(X, keys)
    return _tc_combine(acc)
